# baseline (speedup 1.0000x reference)
import functools
import math

import jax
import jax.numpy as jnp
from jax import lax
from jax.experimental import pallas as pl
from jax.experimental.pallas import tpu as pltpu
from jax.experimental.pallas import tpu_sc as plsc

NUM_BUCKETS = 320
MAX_DISTANCE = 800

T = 2048
H = 16
D = 64
K = 4096
S = 8
R = 2048
_KP = 3968

_NW = 32
_CH = K // _NW


def _bucket_body(out_ref):
    half = NUM_BUCKETS // 2
    small = half // 2
    n = (lax.broadcasted_iota(jnp.int32, (_NW, _CH), 0) * _CH
         + lax.broadcasted_iota(jnp.int32, (_NW, _CH), 1))
    rel = n - (T - 1)
    sign = (rel >= 0).astype(jnp.int32)
    a = jnp.abs(rel)
    is_small = a < small
    ac = jnp.maximum(a, 1)
    log_ratio = jnp.log(ac.astype(jnp.float32) / small) / math.log(MAX_DISTANCE / small)
    log_pos = small + (log_ratio * (half - small)).astype(jnp.int32)
    log_pos = jnp.minimum(log_pos, half - 1)
    bucket = jnp.where(is_small, a, log_pos)
    bucket = bucket + sign * half
    out_ref[...] = jnp.clip(bucket, 0, NUM_BUCKETS - 1)


def _bucket_indices():
    return pl.pallas_call(
        _bucket_body,
        out_shape=jax.ShapeDtypeStruct((_NW, _CH), jnp.int32),
    )()


def _sc_gather_body(table_hbm, idx_hbm, out_hbm, tab_v, idx_v, out_v):
    wid = lax.axis_index("s") * 2 + lax.axis_index("c")
    pltpu.sync_copy(table_hbm, tab_v)
    pltpu.sync_copy(idx_hbm.at[wid], idx_v)
    for j in range(_CH // 16):
        iv = idx_v[pl.ds(j * 16, 16)] * H
        for h in range(H):
            out_v[h, pl.ds(j * 16, 16)] = plsc.load_gather(tab_v, [iv + h])
    for h in range(H):
        pltpu.sync_copy(out_v.at[h], out_hbm.at[(K // _CH) * h + wid])


def _sc_gather(table_flat, idx):
    mesh = plsc.VectorSubcoreMesh(core_axis_name="c", subcore_axis_name="s")
    k = functools.partial(
        pl.kernel,
        out_type=jax.ShapeDtypeStruct((H * K // _CH, _CH), jnp.float32),
        mesh=mesh,
        scratch_types=[
            pltpu.VMEM((NUM_BUCKETS * H,), jnp.float32),
            pltpu.VMEM((_CH,), jnp.int32),
            pltpu.VMEM((H, _CH), jnp.float32),
        ],
        compiler_params=pltpu.CompilerParams(
            use_tc_tiling_on_sc=False, needs_layout_passes=False),
    )(_sc_gather_body)
    return k(table_flat, idx)


def _main_body(line_ref, q_ref, u_ref, w_ref, sc_ref, out_ref, s8_ref, ls_ref):
    h = pl.program_id(0)
    rb = pl.program_id(1)

    @pl.when(rb == 0)
    def _build():
        nrow = K // 128
        slab = line_ref[pl.ds(pl.multiple_of(nrow * h, nrow), nrow), :]
        for c in range(nrow):
            s8_ref[0:1, 128 * c:128 * (c + 1)] = slab[c:c + 1, :]
        lv = s8_ref[0:1, :]
        for s in range(1, S):
            s8_ref[s:s + 1, s:K] = lv[:, 0:K - s]
        for t8 in range(16):
            ls_ref[8 * t8:8 * t8 + 8, 0:_KP] = s8_ref[:, 127 - 8 * t8:127 - 8 * t8 + _KP]

    hm = (lax.broadcasted_iota(jnp.int32, (H, D), 0) == h).astype(jnp.float32)
    uh = jnp.sum(u_ref[...] * hm, axis=0, keepdims=True)
    wh = jnp.sum(w_ref[...] * hm, axis=0, keepdims=True)
    sc = jnp.sum(sc_ref[...]
                 * (lax.broadcasted_iota(jnp.int32, (1, H), 1) == h).astype(jnp.float32))
    qb = q_ref[0]
    gu = 1.0 / (1.0 + jnp.exp(-jnp.sum(qb * uh, axis=1, keepdims=True)))
    gr = 1.0 / (1.0 + jnp.exp(-jnp.sum(qb * wh, axis=1, keepdims=True)))
    f = 1.0 + gu + (1.0 - gu) * (sc * gr)
    for g in range(R // S):
        col = pl.multiple_of(1920 - R * rb - 128 * (g // 16), 128)
        tile = ls_ref[pl.ds(8 * (g % 16), 8), pl.ds(col, T)]
        fg = f[g * S:(g + 1) * S, :]
        out_ref[0, g * S:(g + 1) * S, :] = tile * fg


def _main(line, q3, u, w, scale2):
    grid = (H, T // R)
    return pl.pallas_call(
        _main_body,
        grid=grid,
        in_specs=[
            pl.BlockSpec((H * K // 128, 128), lambda h, r: (0, 0)),
            pl.BlockSpec((1, R, D), lambda h, r: (h, r, 0)),
            pl.BlockSpec((H, D), lambda h, r: (0, 0)),
            pl.BlockSpec((H, D), lambda h, r: (0, 0)),
            pl.BlockSpec((1, H), lambda h, r: (0, 0)),
        ],
        out_specs=pl.BlockSpec((1, R, T), lambda h, r: (h, r, 0)),
        out_shape=jax.ShapeDtypeStruct((H, T, T), jnp.float32),
        scratch_shapes=[
            pltpu.VMEM((S, K), jnp.float32),
            pltpu.VMEM((128, K), jnp.float32),
        ],
    )(line, q3, u, w, scale2)


def kernel(q, seq_len, rel_pos_embed, gate_u_vec, gate_w_vec, gate_scale_scalar):
    B, h, t, d = q.shape
    idx = _bucket_indices()
    line = _sc_gather(rel_pos_embed.reshape(-1), idx)
    out = _main(line, q.reshape(h, t, d), gate_u_vec, gate_w_vec,
                gate_scale_scalar.reshape(1, H))
    return out.reshape(B, h, t, t)


# speedup vs baseline: 1.0079x; 1.0079x over previous
import functools
import math

import jax
import jax.numpy as jnp
from jax import lax
from jax.experimental import pallas as pl
from jax.experimental.pallas import tpu as pltpu
from jax.experimental.pallas import tpu_sc as plsc

NUM_BUCKETS = 320
MAX_DISTANCE = 800

T = 2048
H = 16
D = 64
K = 4096
S = 8
R = 2048
_KP = 3968

_NW = 32
_CH = K // _NW


def _bucket_body(out_ref):
    half = NUM_BUCKETS // 2
    small = half // 2
    n = (lax.broadcasted_iota(jnp.int32, (_NW, _CH), 0) * _CH
         + lax.broadcasted_iota(jnp.int32, (_NW, _CH), 1))
    rel = n - (T - 1)
    sign = (rel >= 0).astype(jnp.int32)
    a = jnp.abs(rel)
    is_small = a < small
    ac = jnp.maximum(a, 1)
    log_ratio = jnp.log(ac.astype(jnp.float32) / small) / math.log(MAX_DISTANCE / small)
    log_pos = small + (log_ratio * (half - small)).astype(jnp.int32)
    log_pos = jnp.minimum(log_pos, half - 1)
    bucket = jnp.where(is_small, a, log_pos)
    bucket = bucket + sign * half
    out_ref[...] = jnp.clip(bucket, 0, NUM_BUCKETS - 1)


def _bucket_indices():
    return pl.pallas_call(
        _bucket_body,
        out_shape=jax.ShapeDtypeStruct((_NW, _CH), jnp.int32),
    )()


def _sc_gather_body(table_hbm, idx_hbm, out_hbm, tab_v, idx_v, out_v):
    wid = lax.axis_index("s") * 2 + lax.axis_index("c")
    pltpu.sync_copy(table_hbm, tab_v)
    pltpu.sync_copy(idx_hbm.at[wid], idx_v)
    for j in range(_CH // 16):
        iv = idx_v[pl.ds(j * 16, 16)] * H
        for h in range(H):
            out_v[h, pl.ds(j * 16, 16)] = plsc.load_gather(tab_v, [iv + h])
    pltpu.sync_copy(out_v, out_hbm.at[:, pl.ds(wid * _CH, _CH)])


def _sc_gather(table_flat, idx):
    mesh = plsc.VectorSubcoreMesh(core_axis_name="c", subcore_axis_name="s")
    k = functools.partial(
        pl.kernel,
        out_type=jax.ShapeDtypeStruct((H, K), jnp.float32),
        mesh=mesh,
        scratch_types=[
            pltpu.VMEM((NUM_BUCKETS * H,), jnp.float32),
            pltpu.VMEM((_CH,), jnp.int32),
            pltpu.VMEM((H, _CH), jnp.float32),
        ],
        compiler_params=pltpu.CompilerParams(
            use_tc_tiling_on_sc=False, needs_layout_passes=False),
    )(_sc_gather_body)
    return k(table_flat, idx)


def _main_body(line_ref, q_ref, u_ref, w_ref, sc_ref, out_ref, s8_ref, ls_ref):
    h = pl.program_id(0)

    hmask = (lax.broadcasted_iota(jnp.int32, (H, K), 0) == h).astype(jnp.float32)
    lv = jnp.sum(line_ref[...] * hmask, axis=0, keepdims=True)
    for s in range(S):
        s8_ref[s:s + 1, s:K] = lv[:, 0:K - s]
    for t8 in range(16):
        ls_ref[8 * t8:8 * t8 + 8, 0:_KP] = s8_ref[:, 127 - 8 * t8:127 - 8 * t8 + _KP]

    hm = (lax.broadcasted_iota(jnp.int32, (H, D), 0) == h).astype(jnp.float32)
    uh = jnp.sum(u_ref[...] * hm, axis=0, keepdims=True)
    wh = jnp.sum(w_ref[...] * hm, axis=0, keepdims=True)
    sc = jnp.sum(sc_ref[...]
                 * (lax.broadcasted_iota(jnp.int32, (1, H), 1) == h).astype(jnp.float32))
    qb = q_ref[0]
    gu = 1.0 / (1.0 + jnp.exp(-jnp.sum(qb * uh, axis=1, keepdims=True)))
    gr = 1.0 / (1.0 + jnp.exp(-jnp.sum(qb * wh, axis=1, keepdims=True)))
    f = 1.0 + gu + (1.0 - gu) * (sc * gr)
    for g in range(R // S):
        col = 1920 - 128 * (g // 16)
        tile = ls_ref[8 * (g % 16):8 * (g % 16) + 8, col:col + T]
        fg = f[g * S:(g + 1) * S, :]
        out_ref[0, g * S:(g + 1) * S, :] = tile * fg


def _main(line, q3, u, w, scale2):
    grid = (H,)
    return pl.pallas_call(
        _main_body,
        grid=grid,
        in_specs=[
            pl.BlockSpec((H, K), lambda h: (0, 0)),
            pl.BlockSpec((1, R, D), lambda h: (h, 0, 0)),
            pl.BlockSpec((H, D), lambda h: (0, 0)),
            pl.BlockSpec((H, D), lambda h: (0, 0)),
            pl.BlockSpec((1, H), lambda h: (0, 0)),
        ],
        out_specs=pl.BlockSpec((1, R, T), lambda h: (h, 0, 0)),
        out_shape=jax.ShapeDtypeStruct((H, T, T), jnp.float32),
        scratch_shapes=[
            pltpu.VMEM((S, K), jnp.float32),
            pltpu.VMEM((128, K), jnp.float32),
        ],
    )(line, q3, u, w, scale2)


def kernel(q, seq_len, rel_pos_embed, gate_u_vec, gate_w_vec, gate_scale_scalar):
    B, h, t, d = q.shape
    idx = _bucket_indices()
    line = _sc_gather(rel_pos_embed.reshape(-1), idx)
    out = _main(line, q.reshape(h, t, d), gate_u_vec, gate_w_vec,
                gate_scale_scalar.reshape(1, H))
    return out.reshape(B, h, t, t)


# speedup vs baseline: 1.0099x; 1.0020x over previous
import functools
import math

import jax
import jax.numpy as jnp
from jax import lax
from jax.experimental import pallas as pl
from jax.experimental.pallas import tpu as pltpu
from jax.experimental.pallas import tpu_sc as plsc

NUM_BUCKETS = 320
MAX_DISTANCE = 800

T = 2048
H = 16
D = 64
K = 4096
S = 8
R = 2048
_KP = 3968

_NW = 32
_CH = K // _NW


def _bucket_body(out_ref):
    half = NUM_BUCKETS // 2
    small = half // 2
    n = (lax.broadcasted_iota(jnp.int32, (_NW, _CH), 0) * _CH
         + lax.broadcasted_iota(jnp.int32, (_NW, _CH), 1))
    rel = n - (T - 1)
    sign = (rel >= 0).astype(jnp.int32)
    a = jnp.abs(rel)
    is_small = a < small
    ac = jnp.maximum(a, 1)
    log_ratio = jnp.log(ac.astype(jnp.float32) / small) / math.log(MAX_DISTANCE / small)
    log_pos = small + (log_ratio * (half - small)).astype(jnp.int32)
    log_pos = jnp.minimum(log_pos, half - 1)
    bucket = jnp.where(is_small, a, log_pos)
    bucket = bucket + sign * half
    out_ref[...] = jnp.clip(bucket, 0, NUM_BUCKETS - 1)


def _bucket_indices():
    return pl.pallas_call(
        _bucket_body,
        out_shape=jax.ShapeDtypeStruct((_NW, _CH), jnp.int32),
    )()


def _sc_gather_body(table_hbm, idx_hbm, out_hbm, tab_v, idx_v, out_v):
    wid = lax.axis_index("s") * 2 + lax.axis_index("c")
    pltpu.sync_copy(table_hbm, tab_v)
    pltpu.sync_copy(idx_hbm.at[wid], idx_v)
    for j in range(_CH // 16):
        iv = idx_v[pl.ds(j * 16, 16)] * H
        for h in range(H):
            fi = iv + h
            out_v[h, pl.ds(j * 16, 16)] = plsc.load_gather(
                tab_v, [lax.shift_right_logical(fi, 7), lax.bitwise_and(fi, 127)])
    pltpu.sync_copy(out_v, out_hbm.at[:, pl.ds(wid * _CH, _CH)])


def _sc_gather(table_flat, idx):
    mesh = plsc.VectorSubcoreMesh(core_axis_name="c", subcore_axis_name="s")
    k = functools.partial(
        pl.kernel,
        out_type=jax.ShapeDtypeStruct((H, K), jnp.float32),
        mesh=mesh,
        scratch_types=[
            pltpu.VMEM((NUM_BUCKETS * H // 128, 128), jnp.float32),
            pltpu.VMEM((_CH,), jnp.int32),
            pltpu.VMEM((H, _CH), jnp.float32),
        ],
        compiler_params=pltpu.CompilerParams(
            use_tc_tiling_on_sc=False, needs_layout_passes=False),
    )(_sc_gather_body)
    return k(table_flat, idx)


def _main_body(line_ref, q_ref, u_ref, w_ref, sc_ref, out_ref, s8_ref, ls_ref):
    h = pl.program_id(0)

    hmask = (lax.broadcasted_iota(jnp.int32, (H, K), 0) == h).astype(jnp.float32)
    lv = jnp.sum(line_ref[...] * hmask, axis=0, keepdims=True)
    for s in range(S):
        s8_ref[s:s + 1, s:K] = lv[:, 0:K - s]
    for t8 in range(16):
        ls_ref[8 * t8:8 * t8 + 8, 0:_KP] = s8_ref[:, 127 - 8 * t8:127 - 8 * t8 + _KP]

    hm = (lax.broadcasted_iota(jnp.int32, (H, D), 0) == h).astype(jnp.float32)
    uh = jnp.sum(u_ref[...] * hm, axis=0, keepdims=True)
    wh = jnp.sum(w_ref[...] * hm, axis=0, keepdims=True)
    sc = jnp.sum(sc_ref[...]
                 * (lax.broadcasted_iota(jnp.int32, (1, H), 1) == h).astype(jnp.float32))
    qb = q_ref[0]
    gu = 1.0 / (1.0 + jnp.exp(-jnp.sum(qb * uh, axis=1, keepdims=True)))
    gr = 1.0 / (1.0 + jnp.exp(-jnp.sum(qb * wh, axis=1, keepdims=True)))
    f = 1.0 + gu + (1.0 - gu) * (sc * gr)
    for g in range(R // S):
        col = 1920 - 128 * (g // 16)
        tile = ls_ref[8 * (g % 16):8 * (g % 16) + 8, col:col + T]
        fg = f[g * S:(g + 1) * S, :]
        out_ref[0, g * S:(g + 1) * S, :] = tile * fg


def _main(line, q3, u, w, scale2):
    grid = (H,)
    return pl.pallas_call(
        _main_body,
        grid=grid,
        in_specs=[
            pl.BlockSpec((H, K), lambda h: (0, 0)),
            pl.BlockSpec((1, R, D), lambda h: (h, 0, 0)),
            pl.BlockSpec((H, D), lambda h: (0, 0)),
            pl.BlockSpec((H, D), lambda h: (0, 0)),
            pl.BlockSpec((1, H), lambda h: (0, 0)),
        ],
        out_specs=pl.BlockSpec((1, R, T), lambda h: (h, 0, 0)),
        out_shape=jax.ShapeDtypeStruct((H, T, T), jnp.float32),
        scratch_shapes=[
            pltpu.VMEM((S, K), jnp.float32),
            pltpu.VMEM((128, K), jnp.float32),
        ],
    )(line, q3, u, w, scale2)


def kernel(q, seq_len, rel_pos_embed, gate_u_vec, gate_w_vec, gate_scale_scalar):
    B, h, t, d = q.shape
    idx = _bucket_indices()
    line = _sc_gather(rel_pos_embed.reshape(-1, 128), idx)
    out = _main(line, q.reshape(h, t, d), gate_u_vec, gate_w_vec,
                gate_scale_scalar.reshape(1, H))
    return out.reshape(B, h, t, t)
